# initial kernel scaffold (unmeasured)
import jax
import jax.numpy as jnp
from jax import lax
from jax.experimental import pallas as pl
from jax.experimental.pallas import tpu as pltpu

B, S, H, D = 1, 1024, 16, 128
SCALE = D ** -0.5


def kernel(Q, K, V):
    def body(q_ref, k_ref, v_ref, out_ref, kv_send, kv_recv, send_sem, recv_sem):
        my_x = lax.axis_index("x")
        my_y = lax.axis_index("y")
        my_z = lax.axis_index("z")
        nbr = (my_x, 1 - my_y, my_z)

        bsem = pltpu.get_barrier_semaphore()
        pl.semaphore_signal(
            bsem, inc=1, device_id=nbr, device_id_type=pl.DeviceIdType.MESH
        )
        pl.semaphore_wait(bsem, 1)

        for h in range(H):
            kv_send[0, h] = k_ref[0, :, h, :].astype(jnp.bfloat16)
            kv_send[1, h] = v_ref[0, :, h, :].astype(jnp.bfloat16)

        rdma = pltpu.make_async_remote_copy(
            src_ref=kv_send,
            dst_ref=kv_recv,
            send_sem=send_sem,
            recv_sem=recv_sem,
            device_id=nbr,
            device_id_type=pl.DeviceIdType.MESH,
        )
        rdma.start()
        rdma.wait()

        dn_qk = (((1,), (1,)), ((), ()))
        dn_pv = (((1,), (0,)), ((), ()))
        for h in range(H):
            q = (q_ref[0, :, h, :] * SCALE).astype(jnp.bfloat16)
            kl = kv_send[0, h]
            vl = kv_send[1, h]
            kr = kv_recv[0, h]
            vr = kv_recv[1, h]
            p1 = jnp.exp(
                lax.dot_general(q, kl, dn_qk, preferred_element_type=jnp.float32)
            )
            l1 = jnp.sum(p1, axis=-1)
            o1 = lax.dot_general(
                p1.astype(jnp.bfloat16), vl, dn_pv,
                preferred_element_type=jnp.float32,
            )
            p2 = jnp.exp(
                lax.dot_general(q, kr, dn_qk, preferred_element_type=jnp.float32)
            )
            l2 = jnp.sum(p2, axis=-1)
            o2 = lax.dot_general(
                p2.astype(jnp.bfloat16), vr, dn_pv,
                preferred_element_type=jnp.float32,
            )
            out_ref[0, :, h, :] = (o1 + o2) / (l1 + l2)[:, None]

    return pl.pallas_call(
        body,
        out_shape=jax.ShapeDtypeStruct((B, S, H, D), jnp.float32),
        in_specs=[pl.BlockSpec(memory_space=pltpu.VMEM)] * 3,
        out_specs=pl.BlockSpec(memory_space=pltpu.VMEM),
        scratch_shapes=[
            pltpu.VMEM((2, H, S, D), jnp.bfloat16),
            pltpu.VMEM((2, H, S, D), jnp.bfloat16),
            pltpu.SemaphoreType.DMA,
            pltpu.SemaphoreType.DMA,
        ],
        compiler_params=pltpu.CompilerParams(collective_id=0),
    )(Q, K, V)


# baseline (device time: 179522 ns/iter reference)
import jax
import jax.numpy as jnp
from jax import lax
from jax.experimental import pallas as pl
from jax.experimental.pallas import tpu as pltpu

B, S, H, D = 1, 1024, 16, 128
SCALE = D ** -0.5


def kernel(Q, K, V):
    def body(q_ref, k_ref, v_ref, out_ref, kv_send, kv_recv, send_sem, recv_sem):
        my_x = lax.axis_index("x")
        my_y = lax.axis_index("y")
        my_z = lax.axis_index("z")
        nbr = (my_x, 1 - my_y, my_z)

        bsem = pltpu.get_barrier_semaphore()
        pl.semaphore_signal(
            bsem, inc=1, device_id=nbr, device_id_type=pl.DeviceIdType.MESH
        )
        pl.semaphore_wait(bsem, 1)

        for h in range(H):
            kv_send[0, h] = k_ref[0, :, h, :].astype(jnp.bfloat16)
            kv_send[1, h] = v_ref[0, :, h, :].astype(jnp.bfloat16)

        rdma = pltpu.make_async_remote_copy(
            src_ref=kv_send,
            dst_ref=kv_recv,
            send_sem=send_sem,
            recv_sem=recv_sem,
            device_id=nbr,
            device_id_type=pl.DeviceIdType.MESH,
        )
        rdma.start()
        rdma.wait()

        dn_qk = (((1,), (1,)), ((), ()))
        dn_pv = (((1,), (0,)), ((), ()))
        for h in range(H):
            q = (q_ref[0, :, h, :] * SCALE).astype(jnp.bfloat16)
            kl = kv_send[0, h]
            vl = kv_send[1, h]
            kr = kv_recv[0, h]
            vr = kv_recv[1, h]
            p1 = jnp.exp(
                lax.dot_general(q, kl, dn_qk, preferred_element_type=jnp.float32)
            )
            l1 = jnp.sum(p1, axis=-1)
            o1 = lax.dot_general(
                p1.astype(jnp.bfloat16), vl, dn_pv,
                preferred_element_type=jnp.float32,
            )
            p2 = jnp.exp(
                lax.dot_general(q, kr, dn_qk, preferred_element_type=jnp.float32)
            )
            l2 = jnp.sum(p2, axis=-1)
            o2 = lax.dot_general(
                p2.astype(jnp.bfloat16), vr, dn_pv,
                preferred_element_type=jnp.float32,
            )
            out_ref[0, :, h, :] = (o1 + o2) / (l1 + l2)[:, None]

    return pl.pallas_call(
        body,
        out_shape=jax.ShapeDtypeStruct((B, S, H, D), jnp.float32),
        in_specs=[pl.BlockSpec(memory_space=pltpu.VMEM)] * 3,
        out_specs=pl.BlockSpec(memory_space=pltpu.VMEM),
        scratch_shapes=[
            pltpu.VMEM((2, H, S, D), jnp.bfloat16),
            pltpu.VMEM((2, H, S, D), jnp.bfloat16),
            pltpu.SemaphoreType.DMA,
            pltpu.SemaphoreType.DMA,
        ],
        compiler_params=pltpu.CompilerParams(
            collective_id=0, vmem_limit_bytes=100 * 1024 * 1024
        ),
    )(Q, K, V)


# device time: 129882 ns/iter; 1.3822x vs baseline; 1.3822x over previous
import jax
import jax.numpy as jnp
from jax import lax
from jax.experimental import pallas as pl
from jax.experimental.pallas import tpu as pltpu

B, S, H, D = 1, 1024, 16, 128
SCALE = D ** -0.5
HALF = S // 2
NC = 4
CS = HALF // NC
HG = H // NC

F32 = jnp.float32
BF16 = jnp.bfloat16
DN_QK = (((1,), (1,)), ((), ()))
DN_PV = (((1,), (0,)), ((), ()))


def kernel(Q, K, V):
    def body(q_ref, k_ref, v_ref, out_ref, kv_s, kv_r, qs, lacc,
             y_send, y_recv, z_send, z_recv):
        my_x = lax.axis_index("x")
        my_y = lax.axis_index("y")
        my_z = lax.axis_index("z")
        nbr_y = (my_x, 1 - my_y, my_z)
        nbr_z = (my_x, my_y, 1 - my_z)

        for h in range(H):
            qs[h] = q_ref[0, :, h, :]
            kv_s[0, h] = k_ref[0, :, h, :]
            kv_s[1, h] = v_ref[0, :, h, :]

        bsem = pltpu.get_barrier_semaphore()
        pl.semaphore_signal(
            bsem, inc=1, device_id=nbr_y, device_id_type=pl.DeviceIdType.MESH
        )
        pl.semaphore_signal(
            bsem, inc=1, device_id=nbr_z, device_id_type=pl.DeviceIdType.MESH
        )
        pl.semaphore_wait(bsem, 2)

        mine0 = my_z * HALF

        def copy(src, dst, ssem, rsem, dev):
            return pltpu.make_async_remote_copy(
                src_ref=src, dst_ref=dst, send_sem=ssem, recv_sem=rsem,
                device_id=dev, device_id_type=pl.DeviceIdType.MESH,
            )

        y_rdmas = []
        for c in range(NC):
            r = copy(
                kv_s.at[:, :, pl.ds(mine0 + c * CS, CS), :],
                kv_r.at[:, :, pl.ds(c * CS, CS), :],
                y_send.at[c], y_recv.at[c], nbr_y,
            )
            r.start()
            y_rdmas.append(r)

        ones = jnp.ones((S, D), BF16)
        ones_half = jnp.ones((HALF, D), BF16)

        def make_attn(kv, rows, aug, mode):

            def f(h, carry):
                q = qs[h]
                k = kv[0, h, rows]
                v = kv[1, h, rows]
                s = lax.dot_general(q, k, DN_QK, preferred_element_type=F32)
                p = jnp.exp(s).astype(BF16)
                vaug = jnp.concatenate([v, aug], axis=1)
                res = lax.dot_general(p, vaug, DN_PV, preferred_element_type=F32)
                o, l = res[:, :D], res[:, D:D + 1]
                if mode == 0:
                    out_ref[h] = o
                    lacc[h] = l
                elif mode == 1:
                    out_ref[h] += o
                    lacc[h] += l
                else:
                    out_ref[h] = (out_ref[h] + o) / (lacc[h] + l)
                return carry

            return f

        local_f = make_attn(kv_s, pl.ds(0, S), ones, mode=0)
        z_rdmas = []
        for c in range(NC):
            lax.fori_loop(c * HG, (c + 1) * HG, local_f, 0)
            y_rdmas[c].wait_recv()
            r = copy(
                kv_r.at[:, :, pl.ds(c * CS, CS), :],
                kv_r.at[:, :, pl.ds(HALF + c * CS, CS), :],
                z_send.at[c], z_recv.at[c], nbr_z,
            )
            r.start()
            z_rdmas.append(r)

        lax.fori_loop(0, H, make_attn(kv_r, pl.ds(0, HALF), ones_half, 1), 0)

        for c in range(NC):
            copy(
                kv_r.at[:, :, pl.ds(c * CS, CS), :],
                kv_r.at[:, :, pl.ds(HALF + c * CS, CS), :],
                z_send.at[c], z_recv.at[c], nbr_z,
            ).wait_recv()
        lax.fori_loop(0, H, make_attn(kv_r, pl.ds(HALF, HALF), ones_half, 2), 0)

        for c in range(NC):
            y_rdmas[c].wait_send()
            z_rdmas[c].wait_send()

    out = pl.pallas_call(
        body,
        out_shape=jax.ShapeDtypeStruct((H, S, D), F32),
        in_specs=[pl.BlockSpec(memory_space=pltpu.VMEM)] * 3,
        out_specs=pl.BlockSpec(memory_space=pltpu.VMEM),
        scratch_shapes=[
            pltpu.VMEM((2, H, S, D), BF16),
            pltpu.VMEM((2, H, S, D), BF16),
            pltpu.VMEM((H, S, D), BF16),
            pltpu.VMEM((H, S, 1), F32),
            pltpu.SemaphoreType.DMA((NC,)),
            pltpu.SemaphoreType.DMA((NC,)),
            pltpu.SemaphoreType.DMA((NC,)),
            pltpu.SemaphoreType.DMA((NC,)),
        ],
        compiler_params=pltpu.CompilerParams(
            collective_id=0, vmem_limit_bytes=100 * 1024 * 1024
        ),
    )(
        (Q * SCALE).astype(BF16),
        K.astype(BF16),
        V.astype(BF16),
    )
    return jnp.transpose(out, (1, 0, 2))[None].astype(jnp.float32)
